# baseline (speedup 1.0000x reference)
import functools

import jax
import jax.numpy as jnp
from jax import lax
from jax.experimental import pallas as pl
from jax.experimental.pallas import tpu as pltpu
from jax.experimental.pallas import tpu_sc as plsc

_LANES = 16


def _pe_add_call(x, enc_weight, cls_row):
    B, T, D = x.shape
    T1 = T + 1
    dtype = x.dtype

    mesh = plsc.VectorSubcoreMesh(core_axis_name="c", subcore_axis_name="s")
    num_workers = mesh.num_cores * mesh.num_subcores
    assert T % num_workers == 0
    rows_per_worker = T // num_workers
    chunk = 16
    assert rows_per_worker % chunk == 0
    n_chunks = rows_per_worker // chunk
    n_vecs = D // _LANES
    nbuf = 2
    n_steps = n_chunks * B

    @functools.partial(
        pl.kernel,
        out_type=jax.ShapeDtypeStruct((T1, B, D), dtype),
        mesh=mesh,
        scratch_types=[
            pltpu.VMEM((chunk, D), dtype),
            pltpu.VMEM((chunk, D), dtype),
            pltpu.VMEM((chunk + 1, D), dtype),
            pltpu.VMEM((chunk + 1, D), dtype),
            pltpu.VMEM((chunk, D), dtype),
            pltpu.VMEM((chunk, D), dtype),
            pltpu.VMEM((1, D), dtype),
            pltpu.SemaphoreType.DMA,
            pltpu.SemaphoreType.DMA,
            pltpu.SemaphoreType.DMA,
            pltpu.SemaphoreType.DMA,
            pltpu.SemaphoreType.DMA,
            pltpu.SemaphoreType.DMA,
        ],
    )
    def pe_add(x_hbm, pe_hbm, cls_hbm, out_hbm,
               xb0, xb1, peb0, peb1, ob0, ob1, cls_v,
               sx0, sx1, so0, so1, sp0, sp1):
        xb = [xb0, xb1]
        peb = [peb0, peb1]
        ob = [ob0, ob1]
        sx = [sx0, sx1]
        so = [so0, so1]
        sp = [sp0, sp1]
        wid = lax.axis_index("s") * mesh.num_cores + lax.axis_index("c")
        base = wid * rows_per_worker

        pe_cd = [None] * n_chunks
        pe_bd = [None] * n_chunks
        x_d = [None] * n_steps
        out_d = [None] * n_steps

        def start_x(s):
            c, b = s // B, s % B
            return pltpu.async_copy(
                x_hbm.at[b, pl.ds(base + c * chunk, chunk)], xb[s % nbuf],
                sx[s % nbuf])

        def start_pe(c):
            pe_cd[c] = pltpu.async_copy(
                pe_hbm.at[pl.ds(base + c * chunk, chunk)],
                peb[c % 2].at[pl.ds(0, chunk)], sp[c % 2])
            pe_bd[c] = pltpu.async_copy(
                pe_hbm.at[pl.ds(base + (c + 1) * chunk, 1)],
                peb[c % 2].at[pl.ds(chunk, 1)], sp[c % 2])

        start_pe(0)
        if n_chunks > 1:
            start_pe(1)
        x_d[0] = start_x(0)

        @pl.when(wid == 0)
        def _():
            pltpu.sync_copy(cls_hbm, cls_v)

        for s in range(n_steps):
            c, b = s // B, s % B
            if s + 1 < n_steps:
                x_d[s + 1] = start_x(s + 1)
            if b == 0:
                pe_cd[c].wait()
                pe_bd[c].wait()
                if 1 <= c and c + 1 < n_chunks:
                    start_pe(c + 1)
            if s == 0:
                @pl.when(wid == 0)
                def _():
                    for j in range(n_vecs):
                        sl = pl.ds(j * _LANES, _LANES)
                        cls_v[0, sl] = cls_v[0, sl] + peb0[0, sl]
                    for b2 in range(B):
                        pltpu.sync_copy(cls_v, out_hbm.at[pl.ds(0, 1), b2])
            x_d[s].wait()
            if s - 2 >= 0:
                out_d[s - 2].wait()
            xv, pv, ov = xb[s % nbuf], peb[c % 2], ob[s % 2]

            def row_add(i, carry):
                for j in range(n_vecs):
                    sl = pl.ds(j * _LANES, _LANES)
                    ov[i, sl] = xv[i, sl] + pv[i + 1, sl]
                return carry

            lax.fori_loop(0, chunk, row_add, 0)

            out_d[s] = pltpu.async_copy(
                ov, out_hbm.at[pl.ds(base + c * chunk + 1, chunk), b],
                so[s % 2])

        for s in range(max(0, n_steps - 2), n_steps):
            out_d[s].wait()

    out_tbd = pe_add(x, enc_weight, cls_row)
    return jnp.transpose(out_tbd, (1, 0, 2))


def kernel(x, enc_weight, cls_tokens_stream, cls_tokens_view, is_stream,
           stream_id, is_view, view_id, use_cls):
    B, T, D = x.shape
    cls_stream = lax.dynamic_slice_in_dim(cls_tokens_stream, stream_id, 1, axis=0)
    cls_view = lax.dynamic_slice_in_dim(cls_tokens_view, view_id, 1, axis=0)
    cls_zero = jnp.zeros((1, 1, D), dtype=x.dtype)
    cls_tok = jnp.where(
        jnp.asarray(is_stream) != 0,
        cls_stream,
        jnp.where(jnp.asarray(is_view) != 0, cls_view, cls_zero),
    )
    cls_tok = cls_tok * jnp.asarray(use_cls, dtype=x.dtype)
    cls_row = cls_tok.reshape(1, D)
    return _pe_add_call(x, enc_weight, cls_row)


# speedup vs baseline: 1.5387x; 1.5387x over previous
import functools

import jax
import jax.numpy as jnp
from jax import lax
from jax.experimental import pallas as pl
from jax.experimental.pallas import tpu as pltpu
from jax.experimental.pallas import tpu_sc as plsc

_LANES = 16


def _pe_add_call(x, enc_weight, cls_row):
    B, T, D = x.shape
    T1 = T + 1
    dtype = x.dtype

    mesh = plsc.VectorSubcoreMesh(core_axis_name="c", subcore_axis_name="s")
    num_workers = mesh.num_cores * mesh.num_subcores
    assert T % num_workers == 0
    rows_per_worker = T // num_workers
    chunk = 16
    assert rows_per_worker % chunk == 0
    n_chunks = rows_per_worker // chunk
    n_vecs = D // _LANES
    nbuf = 2
    n_steps = n_chunks * B

    @functools.partial(
        pl.kernel,
        out_type=jax.ShapeDtypeStruct((T1, B, D), dtype),
        mesh=mesh,
        scratch_types=[
            pltpu.VMEM((chunk, D), dtype),
            pltpu.VMEM((chunk, D), dtype),
            pltpu.VMEM((chunk + 1, D), dtype),
            pltpu.VMEM((chunk + 1, D), dtype),
            pltpu.VMEM((chunk, D), dtype),
            pltpu.VMEM((chunk, D), dtype),
            pltpu.VMEM((1, D), dtype),
            pltpu.SemaphoreType.DMA,
            pltpu.SemaphoreType.DMA,
            pltpu.SemaphoreType.DMA,
            pltpu.SemaphoreType.DMA,
            pltpu.SemaphoreType.DMA,
            pltpu.SemaphoreType.DMA,
        ],
    )
    def pe_add(x_hbm, pe_hbm, cls_hbm, out_hbm,
               xb0, xb1, peb0, peb1, ob0, ob1, cls_v,
               sx0, sx1, so0, so1, sp0, sp1):
        xb = [xb0, xb1]
        peb = [peb0, peb1]
        ob = [ob0, ob1]
        sx = [sx0, sx1]
        so = [so0, so1]
        sp = [sp0, sp1]
        wid = lax.axis_index("s") * mesh.num_cores + lax.axis_index("c")
        base = wid * rows_per_worker

        pe_cd = [None] * n_chunks
        pe_bd = [None] * n_chunks
        x_d = [None] * n_steps
        out_d = [None] * n_steps

        def start_x(s):
            c, b = s // B, s % B
            return pltpu.async_copy(
                x_hbm.at[b, pl.ds(base + c * chunk, chunk)], xb[s % nbuf],
                sx[s % nbuf])

        def start_pe(c):
            pe_cd[c] = pltpu.async_copy(
                pe_hbm.at[pl.ds(base + c * chunk, chunk)],
                peb[c % 2].at[pl.ds(0, chunk)], sp[c % 2])
            pe_bd[c] = pltpu.async_copy(
                pe_hbm.at[pl.ds(base + (c + 1) * chunk, 1)],
                peb[c % 2].at[pl.ds(chunk, 1)], sp[c % 2])

        start_pe(0)
        if n_chunks > 1:
            start_pe(1)
        x_d[0] = start_x(0)

        @pl.when(wid == 0)
        def _():
            pltpu.sync_copy(cls_hbm, cls_v)

        for s in range(n_steps):
            c, b = s // B, s % B
            if s + 1 < n_steps:
                x_d[s + 1] = start_x(s + 1)
            if b == 0:
                pe_cd[c].wait()
                pe_bd[c].wait()
                if 1 <= c and c + 1 < n_chunks:
                    start_pe(c + 1)
            if s == 0:
                @pl.when(wid == 0)
                def _():
                    for j in range(n_vecs):
                        sl = pl.ds(j * _LANES, _LANES)
                        cls_v[0, sl] = cls_v[0, sl] + peb0[0, sl]
                    for b2 in range(B):
                        pltpu.sync_copy(cls_v, out_hbm.at[pl.ds(0, 1), b2])
            x_d[s].wait()
            if s - 2 >= 0:
                out_d[s - 2].wait()
            xv, pv, ov = xb[s % nbuf], peb[c % 2], ob[s % 2]

            def col_add(j, carry):
                sl = pl.ds(j * _LANES, _LANES)
                for i in range(chunk):
                    ov[i, sl] = xv[i, sl] + pv[i + 1, sl]
                return carry

            lax.fori_loop(0, n_vecs, col_add, 0)

            out_d[s] = pltpu.async_copy(
                ov, out_hbm.at[pl.ds(base + c * chunk + 1, chunk), b],
                so[s % 2])

        for s in range(max(0, n_steps - 2), n_steps):
            out_d[s].wait()

    out_tbd = pe_add(x, enc_weight, cls_row)
    return jnp.transpose(out_tbd, (1, 0, 2))


def kernel(x, enc_weight, cls_tokens_stream, cls_tokens_view, is_stream,
           stream_id, is_view, view_id, use_cls):
    B, T, D = x.shape
    cls_stream = lax.dynamic_slice_in_dim(cls_tokens_stream, stream_id, 1, axis=0)
    cls_view = lax.dynamic_slice_in_dim(cls_tokens_view, view_id, 1, axis=0)
    cls_zero = jnp.zeros((1, 1, D), dtype=x.dtype)
    cls_tok = jnp.where(
        jnp.asarray(is_stream) != 0,
        cls_stream,
        jnp.where(jnp.asarray(is_view) != 0, cls_view, cls_zero),
    )
    cls_tok = cls_tok * jnp.asarray(use_cls, dtype=x.dtype)
    cls_row = cls_tok.reshape(1, D)
    return _pe_add_call(x, enc_weight, cls_row)


# speedup vs baseline: 1.7636x; 1.1461x over previous
import functools

import jax
import jax.numpy as jnp
from jax import lax
from jax.experimental import pallas as pl
from jax.experimental.pallas import tpu as pltpu
from jax.experimental.pallas import tpu_sc as plsc

_LANES = 16


def _pe_add_call(x, enc_weight, cls_row):
    B, T, D = x.shape
    T1 = T + 1
    dtype = x.dtype

    mesh = plsc.VectorSubcoreMesh(core_axis_name="c", subcore_axis_name="s")
    num_workers = mesh.num_cores * mesh.num_subcores
    assert T % num_workers == 0
    rows_per_worker = T // num_workers
    chunk = 16
    assert rows_per_worker % chunk == 0
    n_chunks = rows_per_worker // chunk
    n_vecs = D // _LANES
    nbuf = 2
    n_steps = n_chunks * B

    @functools.partial(
        pl.kernel,
        out_type=jax.ShapeDtypeStruct((T1, B, D), dtype),
        mesh=mesh,
        scratch_types=[
            pltpu.VMEM((chunk, D), dtype),
            pltpu.VMEM((chunk, D), dtype),
            pltpu.VMEM((chunk + 1, D), dtype),
            pltpu.VMEM((chunk + 1, D), dtype),
            pltpu.VMEM((chunk, D), dtype),
            pltpu.VMEM((chunk, D), dtype),
            pltpu.VMEM((1, D), dtype),
            pltpu.SemaphoreType.DMA,
            pltpu.SemaphoreType.DMA,
            pltpu.SemaphoreType.DMA,
            pltpu.SemaphoreType.DMA,
            pltpu.SemaphoreType.DMA,
            pltpu.SemaphoreType.DMA,
        ],
    )
    def pe_add(x_hbm, pe_hbm, cls_hbm, out_hbm,
               xb0, xb1, peb0, peb1, ob0, ob1, cls_v,
               sx0, sx1, so0, so1, sp0, sp1):
        xb = [xb0, xb1]
        peb = [peb0, peb1]
        ob = [ob0, ob1]
        sx = [sx0, sx1]
        so = [so0, so1]
        sp = [sp0, sp1]
        wid = lax.axis_index("s") * mesh.num_cores + lax.axis_index("c")
        base = wid * rows_per_worker

        pe_cd = [None] * n_chunks
        pe_bd = [None] * n_chunks
        x_d = [None] * n_steps
        out_d = [None] * n_steps

        def start_x(s):
            c, b = s // B, s % B
            return pltpu.async_copy(
                x_hbm.at[b, pl.ds(base + c * chunk, chunk)], xb[s % nbuf],
                sx[s % nbuf])

        def start_pe(c):
            pe_cd[c] = pltpu.async_copy(
                pe_hbm.at[pl.ds(base + c * chunk, chunk)],
                peb[c % 2].at[pl.ds(0, chunk)], sp[c % 2])
            pe_bd[c] = pltpu.async_copy(
                pe_hbm.at[pl.ds(base + (c + 1) * chunk, 1)],
                peb[c % 2].at[pl.ds(chunk, 1)], sp[c % 2])

        start_pe(0)
        if n_chunks > 1:
            start_pe(1)
        x_d[0] = start_x(0)

        @pl.when(wid == 0)
        def _():
            pltpu.sync_copy(cls_hbm, cls_v)

        for s in range(n_steps):
            c, b = s // B, s % B
            if s + 1 < n_steps:
                x_d[s + 1] = start_x(s + 1)
            if b == 0:
                pe_cd[c].wait()
                pe_bd[c].wait()
                if 1 <= c and c + 1 < n_chunks:
                    start_pe(c + 1)
            if s == 0:
                @pl.when(wid == 0)
                def _():
                    for j in range(n_vecs):
                        sl = pl.ds(j * _LANES, _LANES)
                        cls_v[0, sl] = cls_v[0, sl] + peb0[0, sl]
                    for b2 in range(B):
                        pltpu.sync_copy(cls_v, out_hbm.at[pl.ds(0, 1), b2])
            x_d[s].wait()
            if s - 2 >= 0:
                out_d[s - 2].wait()
            xv, pv, ov = xb[s % nbuf], peb[c % 2], ob[s % 2]

            @plsc.parallel_loop(0, n_vecs, unroll=2)
            def col_add(j):
                sl = pl.ds(j * _LANES, _LANES)
                for i in range(chunk):
                    ov[i, sl] = xv[i, sl] + pv[i + 1, sl]

            out_d[s] = pltpu.async_copy(
                ov, out_hbm.at[pl.ds(base + c * chunk + 1, chunk), b],
                so[s % 2])

        for s in range(max(0, n_steps - 2), n_steps):
            out_d[s].wait()

    out_tbd = pe_add(x, enc_weight, cls_row)
    return jnp.transpose(out_tbd, (1, 0, 2))


def kernel(x, enc_weight, cls_tokens_stream, cls_tokens_view, is_stream,
           stream_id, is_view, view_id, use_cls):
    B, T, D = x.shape
    cls_stream = lax.dynamic_slice_in_dim(cls_tokens_stream, stream_id, 1, axis=0)
    cls_view = lax.dynamic_slice_in_dim(cls_tokens_view, view_id, 1, axis=0)
    cls_zero = jnp.zeros((1, 1, D), dtype=x.dtype)
    cls_tok = jnp.where(
        jnp.asarray(is_stream) != 0,
        cls_stream,
        jnp.where(jnp.asarray(is_view) != 0, cls_view, cls_zero),
    )
    cls_tok = cls_tok * jnp.asarray(use_cls, dtype=x.dtype)
    cls_row = cls_tok.reshape(1, D)
    return _pe_add_call(x, enc_weight, cls_row)
